# Initial kernel scaffold; baseline (speedup 1.0000x reference)
import jax
import jax.numpy as jnp
from jax.experimental import pallas as pl


def kernel(x, Wg, W1, W2):
    raise NotImplementedError("write your pallas kernel here")


# speedup vs baseline: 1.5595x; 1.5595x over previous
import functools
import math

import jax
import jax.numpy as jnp
from jax.experimental import pallas as pl
from jax.experimental.pallas import tpu as pltpu

D_MODEL = 1024
HIDDEN = 2048
NUM_E = 8
TOP_K = 2
SEQ = 4096
CAP = int(math.ceil(1.25 * (SEQ * TOP_K) / NUM_E))

_R = 32
_C = 128


def _excl_prefix_flat(v2d):
    ci = jax.lax.broadcasted_iota(jnp.int32, (_C, _C), 0)
    cj = jax.lax.broadcasted_iota(jnp.int32, (_C, _C), 1)
    tlow_c = (ci < cj).astype(jnp.float32)
    within = jax.lax.dot_general(v2d, tlow_c, (((1,), (0,)), ((), ())),
                                 preferred_element_type=jnp.float32)
    rowsum = jnp.sum(v2d, axis=1, keepdims=True)
    ri = jax.lax.broadcasted_iota(jnp.int32, (_R, _R), 0)
    rj = jax.lax.broadcasted_iota(jnp.int32, (_R, _R), 1)
    tlow_r = (ri < rj).astype(jnp.float32)
    rowoff = jax.lax.dot_general(rowsum.reshape(1, _R), tlow_r,
                                 (((1,), (0,)), ((), ())),
                                 preferred_element_type=jnp.float32)
    return within + rowoff.reshape(_R, 1)


def _router_body(x_ref, wg_ref, g_ref, aux_ref):
    x = x_ref[...]
    wg = wg_ref[...]
    logits = jax.lax.dot_general(x, wg, (((1,), (1,)), ((), ())),
                                 preferred_element_type=jnp.float32)
    m = jnp.max(logits, axis=1, keepdims=True)
    ex = jnp.exp(logits - m)
    scores = ex / jnp.sum(ex, axis=1, keepdims=True)

    lane = jax.lax.broadcasted_iota(jnp.int32, (SEQ, NUM_E), 1)
    v1 = jnp.max(scores, axis=1)
    i1 = jnp.argmax(scores, axis=1).astype(jnp.int32)
    masked = jnp.where(lane == i1[:, None], -jnp.inf, scores)
    v2 = jnp.max(masked, axis=1)
    i2 = jnp.argmax(masked, axis=1).astype(jnp.int32)

    i1r = i1.reshape(_R, _C)
    i2r = i2.reshape(_R, _C)
    v1r = v1.reshape(_R, _C)
    v2r = v2.reshape(_R, _C)
    sb1 = jax.lax.bitcast_convert_type(v1r, jnp.int32)
    sb2 = jax.lax.bitcast_convert_type(v2r, jnp.int32)

    gate_cols = []
    load = []
    for e in range(NUM_E):
        m1 = i1r == e
        m2 = i2r == e
        load.append((jnp.sum(m1.astype(jnp.float32))
                     + jnp.sum(m2.astype(jnp.float32))))

        def bit_step(b, t, m1=m1, m2=m2, sb1=sb1, sb2=sb2):
            t2 = t | (1 << (30 - b))
            cnt = (jnp.sum(jnp.where(m1 & (sb1 >= t2), 1.0, 0.0))
                   + jnp.sum(jnp.where(m2 & (sb2 >= t2), 1.0, 0.0)))
            return jnp.where(cnt >= CAP, t2, t)

        t_e = jax.lax.fori_loop(0, 31, bit_step, jnp.int32(0))

        gt1 = m1 & (sb1 > t_e)
        gt2 = m2 & (sb2 > t_e)
        eq1 = m1 & (sb1 == t_e)
        eq2 = m2 & (sb2 == t_e)
        n_gt = (jnp.sum(gt1.astype(jnp.float32))
                + jnp.sum(gt2.astype(jnp.float32)))
        rem = jnp.float32(CAP) - n_gt
        eqc = eq1.astype(jnp.float32) + eq2.astype(jnp.float32)
        ex_tok = _excl_prefix_flat(eqc)
        tie1 = ex_tok
        tie2 = ex_tok + eq1.astype(jnp.float32)
        kept1 = gt1 | (eq1 & (tie1 < rem))
        kept2 = gt2 | (eq2 & (tie2 < rem))

        ge = (jnp.where(kept1, v1r, 0.0) + jnp.where(kept2, v2r, 0.0))
        gate_cols.append(ge.reshape(SEQ, 1))

    g_ref[...] = jnp.concatenate(gate_cols, axis=1)

    importance = jnp.sum(scores, axis=0)
    load_v = jnp.stack(load)
    aux = jnp.sum(importance * load_v) * (NUM_E / (SEQ * SEQ))
    aux_ref[...] = aux.reshape(1, 1)


def _mlp_body(xb_ref, w1_ref, w2_ref, g_ref, out_ref):
    e = pl.program_id(1)
    xb = xb_ref[...]
    w1 = w1_ref[0]
    w2 = w2_ref[0]
    h = jax.lax.dot_general(xb, w1, (((1,), (1,)), ((), ())),
                            preferred_element_type=jnp.float32)
    h = 0.5 * h * (1.0 + jax.lax.erf(h * 0.7071067811865476))
    hb = h.astype(jnp.bfloat16)
    y = jax.lax.dot_general(hb, w2, (((1,), (1,)), ((), ())),
                            preferred_element_type=jnp.float32)
    gcol = g_ref[0, 0, :]
    contrib = y * gcol[:, None]

    @pl.when(e == 0)
    def _init():
        out_ref[...] = contrib

    @pl.when(e > 0)
    def _acc():
        out_ref[...] += contrib


_BS = 512
_NS_T = SEQ // _BS


@jax.jit
def kernel(x, Wg, W1, W2):
    g, aux = pl.pallas_call(
        _router_body,
        out_shape=(
            jax.ShapeDtypeStruct((SEQ, NUM_E), jnp.float32),
            jax.ShapeDtypeStruct((1, 1), jnp.float32),
        ),
        in_specs=[
            pl.BlockSpec((SEQ, D_MODEL), lambda: (0, 0)),
            pl.BlockSpec((NUM_E, D_MODEL), lambda: (0, 0)),
        ],
        out_specs=(
            pl.BlockSpec((SEQ, NUM_E), lambda: (0, 0)),
            pl.BlockSpec((1, 1), lambda: (0, 0)),
        ),
    )(x, Wg)

    gt = g.T.reshape(NUM_E, 1, SEQ)
    xb = x.astype(jnp.bfloat16)
    w1b = W1.astype(jnp.bfloat16)
    w2b = W2.astype(jnp.bfloat16)

    out = pl.pallas_call(
        _mlp_body,
        grid=(_NS_T, NUM_E),
        out_shape=jax.ShapeDtypeStruct((SEQ, D_MODEL), jnp.float32),
        in_specs=[
            pl.BlockSpec((_BS, D_MODEL), lambda s, e: (s, 0)),
            pl.BlockSpec((1, HIDDEN, D_MODEL), lambda s, e: (e, 0, 0)),
            pl.BlockSpec((1, D_MODEL, HIDDEN), lambda s, e: (e, 0, 0)),
            pl.BlockSpec((1, 1, _BS), lambda s, e: (e, 0, s)),
        ],
        out_specs=pl.BlockSpec((_BS, D_MODEL), lambda s, e: (s, 0)),
    )(xb, w1b, w2b, gt)

    return out, aux.reshape(())


# speedup vs baseline: 1.5656x; 1.0039x over previous
import functools
import math

import jax
import jax.numpy as jnp
from jax import lax
from jax.experimental import pallas as pl
from jax.experimental.pallas import tpu as pltpu
from jax.experimental.pallas import tpu_sc as plsc

D_MODEL = 1024
HIDDEN = 2048
NUM_E = 8
TOP_K = 2
SEQ = 4096
CAP = int(math.ceil(1.25 * (SEQ * TOP_K) / NUM_E))
NROWS = NUM_E * CAP
TRASH = NROWS
NTOK = NROWS + 32

_R = 32
_C = 128


def _excl_prefix_flat(v2d):
    ci = lax.broadcasted_iota(jnp.int32, (_C, _C), 0)
    cj = lax.broadcasted_iota(jnp.int32, (_C, _C), 1)
    tlow_c = (ci < cj).astype(jnp.float32)
    within = lax.dot_general(v2d, tlow_c, (((1,), (0,)), ((), ())),
                             preferred_element_type=jnp.float32)
    rowsum = jnp.sum(v2d, axis=1, keepdims=True)
    ri = lax.broadcasted_iota(jnp.int32, (_R, _R), 0)
    rj = lax.broadcasted_iota(jnp.int32, (_R, _R), 1)
    tlow_r = (ri < rj).astype(jnp.float32)
    rowoff = lax.dot_general(rowsum.reshape(1, _R), tlow_r,
                             (((1,), (0,)), ((), ())),
                             preferred_element_type=jnp.float32)
    return within + rowoff.reshape(_R, 1)


def _score_body(x_ref, wg_ref, s_ref):
    x = x_ref[...]
    wg = wg_ref[...]
    logits = lax.dot_general(x, wg, (((1,), (1,)), ((), ())),
                             preferred_element_type=jnp.float32)
    m = jnp.max(logits, axis=1, keepdims=True)
    ex = jnp.exp(logits - m)
    s_ref[...] = ex / jnp.sum(ex, axis=1, keepdims=True)


def _route_body(s_ref, dest_ref, cidx_ref, g_ref, aux_ref):
    scores = s_ref[...]

    lane = lax.broadcasted_iota(jnp.int32, (SEQ, NUM_E), 1)
    v1 = jnp.max(scores, axis=1)
    i1 = jnp.argmax(scores, axis=1).astype(jnp.int32)
    masked = jnp.where(lane == i1[:, None], -jnp.inf, scores)
    v2 = jnp.max(masked, axis=1)
    i2 = jnp.argmax(masked, axis=1).astype(jnp.int32)

    i1r = i1.reshape(_R, _C)
    i2r = i2.reshape(_R, _C)
    v1r = v1.reshape(_R, _C)
    v2r = v2.reshape(_R, _C)
    sb1 = lax.bitcast_convert_type(v1r, jnp.int32)
    sb2 = lax.bitcast_convert_type(v2r, jnp.int32)

    dest1 = jnp.full((_R, _C), TRASH, jnp.int32)
    dest2 = jnp.full((_R, _C), TRASH, jnp.int32)
    cidx1 = jnp.zeros((_R, _C), jnp.int32)
    cidx2 = jnp.zeros((_R, _C), jnp.int32)
    g1 = jnp.zeros((_R, _C), jnp.float32)
    g2 = jnp.zeros((_R, _C), jnp.float32)
    load = []
    for e in range(NUM_E):
        m1 = i1r == e
        m2 = i2r == e
        load.append(jnp.sum(m1.astype(jnp.float32))
                    + jnp.sum(m2.astype(jnp.float32)))

        def bit_step(b, t, m1=m1, m2=m2, sb1=sb1, sb2=sb2):
            t2 = t | (1 << (30 - b))
            cnt = (jnp.sum(jnp.where(m1 & (sb1 >= t2), 1.0, 0.0))
                   + jnp.sum(jnp.where(m2 & (sb2 >= t2), 1.0, 0.0)))
            return jnp.where(cnt >= CAP, t2, t)

        t_e = lax.fori_loop(0, 31, bit_step, jnp.int32(0))

        gt1 = m1 & (sb1 > t_e)
        gt2 = m2 & (sb2 > t_e)
        eq1 = m1 & (sb1 == t_e)
        eq2 = m2 & (sb2 == t_e)
        n_gt = (jnp.sum(gt1.astype(jnp.float32))
                + jnp.sum(gt2.astype(jnp.float32)))
        rem = jnp.float32(CAP) - n_gt
        eqc = eq1.astype(jnp.float32) + eq2.astype(jnp.float32)
        ex_eq = _excl_prefix_flat(eqc)
        kept1 = gt1 | (eq1 & (ex_eq < rem))
        kept2 = gt2 | (eq2 & ((ex_eq + eq1.astype(jnp.float32)) < rem))

        keptc = kept1.astype(jnp.float32) + kept2.astype(jnp.float32)
        ex_k = _excl_prefix_flat(keptc)
        pos1 = ex_k.astype(jnp.int32)
        pos2 = (ex_k + kept1.astype(jnp.float32)).astype(jnp.int32)

        dest1 = jnp.where(kept1, e * CAP + pos1, dest1)
        dest2 = jnp.where(kept2, e * CAP + pos2, dest2)
        cidx1 = jnp.where(kept1, e * CAP + pos1, cidx1)
        cidx2 = jnp.where(kept2, e * CAP + pos2, cidx2)
        g1 = jnp.where(kept1, v1r, g1)
        g2 = jnp.where(kept2, v2r, g2)

    dest_ref[...] = jnp.stack([dest1.reshape(SEQ), dest2.reshape(SEQ)])
    cidx_ref[...] = jnp.stack([cidx1.reshape(SEQ), cidx2.reshape(SEQ)])
    g_ref[...] = jnp.stack([g1.reshape(SEQ), g2.reshape(SEQ)]).reshape(
        TOP_K, 1, SEQ)

    importance = jnp.sum(scores, axis=0)
    load_v = jnp.stack(load)
    aux = jnp.sum(importance * load_v) * (NUM_E / (SEQ * SEQ))
    aux_ref[...] = aux.reshape(1, 1)


_SC_INFO = plsc.get_sparse_core_info()
_NC = _SC_INFO.num_cores
_NSUB = _SC_INFO.num_subcores
_NW = _NC * _NSUB
_RPW = NROWS // _NW
_GPW = (SEQ * TOP_K) // _NW
_CH = 64


def _dispatch_body(x_hbm, destf_hbm, xbuf_hbm,
                   dest_v, tok_v, shared_tok, idx_c, rows_v, sem):
    cid = lax.axis_index("c")
    sid = lax.axis_index("s")
    wid = sid * _NC + cid

    @pl.when(sid == 0)
    def _build_toklist():
        pltpu.sync_copy(destf_hbm, dest_v)

        def zinit(i, c):
            tok_v[pl.ds(i * 16, 16)] = jnp.zeros((16,), jnp.int32)
            return c

        lax.fori_loop(0, NTOK // 16, zinit, 0)

        def scat(i, c):
            idx = dest_v[pl.ds(i * 16, 16)]
            tok = (lax.rem(i, SEQ // 16) * 16
                   + lax.broadcasted_iota(jnp.int32, (16,), 0))
            plsc.store_scatter(tok_v, [idx], tok)
            return c

        lax.fori_loop(0, (SEQ * TOP_K) // 16, scat, 0)
        pltpu.sync_copy(tok_v, shared_tok)

    plsc.subcore_barrier()

    base = wid * _RPW
    for ch in range(_RPW // _CH):
        off = base + ch * _CH
        pltpu.sync_copy(shared_tok.at[pl.ds(off, _CH)], idx_c)
        pltpu.async_copy(x_hbm.at[idx_c], rows_v, sem).wait()
        pltpu.sync_copy(rows_v, xbuf_hbm.at[pl.ds(off, _CH)])


def _gatherout_body(ybuf_hbm, cidxf_hbm, yg_hbm, idx_c, rows_v, sem):
    cid = lax.axis_index("c")
    sid = lax.axis_index("s")
    wid = sid * _NC + cid
    base = wid * _GPW
    for ch in range(_GPW // _CH):
        off = base + ch * _CH
        pltpu.sync_copy(cidxf_hbm.at[pl.ds(off, _CH)], idx_c)
        pltpu.async_copy(ybuf_hbm.at[idx_c], rows_v, sem).wait()
        pltpu.sync_copy(rows_v, yg_hbm.at[pl.ds(off, _CH)])


def _mlp_body(xb_ref, w1_ref, w2_ref, out_ref):
    xb = xb_ref[...].astype(jnp.bfloat16)
    w1 = w1_ref[0]
    w2 = w2_ref[0]
    h = lax.dot_general(xb, w1, (((1,), (1,)), ((), ())),
                        preferred_element_type=jnp.float32)
    h = 0.5 * h * (1.0 + lax.erf(h * 0.7071067811865476))
    y = lax.dot_general(h.astype(jnp.bfloat16), w2, (((1,), (1,)), ((), ())),
                        preferred_element_type=jnp.float32)
    out_ref[...] = y


def _combine_body(yg_ref, g_ref, out_ref):
    y = yg_ref[...]
    g = g_ref[...]
    out_ref[...] = g[0, 0, :, None] * y[0] + g[1, 0, :, None] * y[1]


_BS = 640
_CBS = 512


@jax.jit
def kernel(x, Wg, W1, W2):
    scores = pl.pallas_call(
        _score_body,
        grid=(SEQ // _CBS,),
        out_shape=jax.ShapeDtypeStruct((SEQ, NUM_E), jnp.float32),
        in_specs=[
            pl.BlockSpec((_CBS, D_MODEL), lambda s: (s, 0)),
            pl.BlockSpec((NUM_E, D_MODEL), lambda s: (0, 0)),
        ],
        out_specs=pl.BlockSpec((_CBS, NUM_E), lambda s: (s, 0)),
    )(x, Wg)

    dest, cidx, g, aux = pl.pallas_call(
        _route_body,
        out_shape=(
            jax.ShapeDtypeStruct((TOP_K, SEQ), jnp.int32),
            jax.ShapeDtypeStruct((TOP_K, SEQ), jnp.int32),
            jax.ShapeDtypeStruct((TOP_K, 1, SEQ), jnp.float32),
            jax.ShapeDtypeStruct((1, 1), jnp.float32),
        ),
        in_specs=[
            pl.BlockSpec((SEQ, NUM_E), lambda: (0, 0)),
        ],
        out_specs=(
            pl.BlockSpec((TOP_K, SEQ), lambda: (0, 0)),
            pl.BlockSpec((TOP_K, SEQ), lambda: (0, 0)),
            pl.BlockSpec((TOP_K, 1, SEQ), lambda: (0, 0, 0)),
            pl.BlockSpec((1, 1), lambda: (0, 0)),
        ),
    )(scores)

    mesh = plsc.VectorSubcoreMesh(core_axis_name="c", subcore_axis_name="s")

    sc_params = pltpu.CompilerParams(needs_layout_passes=False)
    xbuf = pl.kernel(
        _dispatch_body,
        out_type=jax.ShapeDtypeStruct((NROWS, D_MODEL), jnp.float32),
        mesh=mesh,
        compiler_params=sc_params,
        scratch_types=[
            pltpu.VMEM((SEQ * TOP_K,), jnp.int32),
            pltpu.VMEM((NTOK,), jnp.int32),
            pltpu.VMEM_SHARED((NTOK,), jnp.int32),
            pltpu.VMEM((_CH,), jnp.int32),
            pltpu.VMEM((_CH, D_MODEL), jnp.float32),
            pltpu.SemaphoreType.DMA,
        ],
    )(x, dest.reshape(SEQ * TOP_K))

    w1b = W1.astype(jnp.bfloat16)
    w2b = W2.astype(jnp.bfloat16)
    ybuf = pl.pallas_call(
        _mlp_body,
        grid=(NUM_E, CAP // _BS),
        out_shape=jax.ShapeDtypeStruct((NROWS, D_MODEL), jnp.float32),
        in_specs=[
            pl.BlockSpec((_BS, D_MODEL), lambda e, t: (e * (CAP // _BS) + t, 0)),
            pl.BlockSpec((1, HIDDEN, D_MODEL), lambda e, t: (e, 0, 0)),
            pl.BlockSpec((1, D_MODEL, HIDDEN), lambda e, t: (e, 0, 0)),
        ],
        out_specs=pl.BlockSpec((_BS, D_MODEL),
                               lambda e, t: (e * (CAP // _BS) + t, 0)),
    )(xbuf, w1b, w2b)

    yg = pl.kernel(
        _gatherout_body,
        out_type=jax.ShapeDtypeStruct((SEQ * TOP_K, D_MODEL), jnp.float32),
        mesh=mesh,
        compiler_params=sc_params,
        scratch_types=[
            pltpu.VMEM((_CH,), jnp.int32),
            pltpu.VMEM((_CH, D_MODEL), jnp.float32),
            pltpu.SemaphoreType.DMA,
        ],
    )(ybuf, cidx.reshape(SEQ * TOP_K))

    out = pl.pallas_call(
        _combine_body,
        grid=(SEQ // _CBS,),
        out_shape=jax.ShapeDtypeStruct((SEQ, D_MODEL), jnp.float32),
        in_specs=[
            pl.BlockSpec((TOP_K, _CBS, D_MODEL), lambda s: (0, s, 0)),
            pl.BlockSpec((TOP_K, 1, _CBS), lambda s: (0, 0, s)),
        ],
        out_specs=pl.BlockSpec((_CBS, D_MODEL), lambda s: (s, 0)),
    )(yg.reshape(TOP_K, SEQ, D_MODEL), g)

    return out, aux.reshape(())


# speedup vs baseline: 1.6726x; 1.0684x over previous
import functools
import math

import jax
import jax.numpy as jnp
from jax import lax
from jax.experimental import pallas as pl
from jax.experimental.pallas import tpu as pltpu
from jax.experimental.pallas import tpu_sc as plsc

D_MODEL = 1024
HIDDEN = 2048
NUM_E = 8
TOP_K = 2
SEQ = 4096
CAP = int(math.ceil(1.25 * (SEQ * TOP_K) / NUM_E))
NROWS = NUM_E * CAP
TRASH = NROWS
NTOK = NROWS + 256

_R = 32
_C = 128


def _excl_prefix_flat(v2d):
    ci = lax.broadcasted_iota(jnp.int32, (_C, _C), 0)
    cj = lax.broadcasted_iota(jnp.int32, (_C, _C), 1)
    tlow_c = (ci < cj).astype(jnp.float32)
    within = lax.dot_general(v2d, tlow_c, (((1,), (0,)), ((), ())),
                             preferred_element_type=jnp.float32)
    rowsum = jnp.sum(v2d, axis=1, keepdims=True)
    ri = lax.broadcasted_iota(jnp.int32, (_R, _R), 0)
    rj = lax.broadcasted_iota(jnp.int32, (_R, _R), 1)
    tlow_r = (ri < rj).astype(jnp.float32)
    rowoff = lax.dot_general(rowsum.reshape(1, _R), tlow_r,
                             (((1,), (0,)), ((), ())),
                             preferred_element_type=jnp.float32)
    return within + rowoff.reshape(_R, 1)


def _score_body(x_ref, wg_ref, s_ref):
    x = x_ref[...]
    wg = wg_ref[...]
    logits = lax.dot_general(x, wg, (((1,), (1,)), ((), ())),
                             preferred_element_type=jnp.float32)
    m = jnp.max(logits, axis=1, keepdims=True)
    ex = jnp.exp(logits - m)
    s_ref[...] = ex / jnp.sum(ex, axis=1, keepdims=True)


def _route_body(s_ref, dest_ref, cidx_ref, g_ref, aux_ref):
    scores = s_ref[...]

    lane = lax.broadcasted_iota(jnp.int32, (SEQ, NUM_E), 1)
    v1 = jnp.max(scores, axis=1)
    i1 = jnp.argmax(scores, axis=1).astype(jnp.int32)
    masked = jnp.where(lane == i1[:, None], -jnp.inf, scores)
    v2 = jnp.max(masked, axis=1)
    i2 = jnp.argmax(masked, axis=1).astype(jnp.int32)

    i1r = i1.reshape(_R, _C)
    i2r = i2.reshape(_R, _C)
    v1r = v1.reshape(_R, _C)
    v2r = v2.reshape(_R, _C)
    sb1 = lax.bitcast_convert_type(v1r, jnp.int32)
    sb2 = lax.bitcast_convert_type(v2r, jnp.int32)

    dest1 = jnp.full((_R, _C), TRASH, jnp.int32)
    dest2 = jnp.full((_R, _C), TRASH, jnp.int32)
    cidx1 = jnp.zeros((_R, _C), jnp.int32)
    cidx2 = jnp.zeros((_R, _C), jnp.int32)
    g1 = jnp.zeros((_R, _C), jnp.float32)
    g2 = jnp.zeros((_R, _C), jnp.float32)
    load = []
    for e in range(NUM_E):
        m1 = i1r == e
        m2 = i2r == e
        load.append(jnp.sum(m1.astype(jnp.float32))
                    + jnp.sum(m2.astype(jnp.float32)))

        def bit_step(b, t, m1=m1, m2=m2, sb1=sb1, sb2=sb2):
            t2 = t | (1 << (30 - b))
            cnt = (jnp.sum(jnp.where(m1 & (sb1 >= t2), 1.0, 0.0))
                   + jnp.sum(jnp.where(m2 & (sb2 >= t2), 1.0, 0.0)))
            return jnp.where(cnt >= CAP, t2, t)

        t_e = lax.fori_loop(0, 31, bit_step, jnp.int32(0))

        gt1 = m1 & (sb1 > t_e)
        gt2 = m2 & (sb2 > t_e)
        eq1 = m1 & (sb1 == t_e)
        eq2 = m2 & (sb2 == t_e)
        n_gt = (jnp.sum(gt1.astype(jnp.float32))
                + jnp.sum(gt2.astype(jnp.float32)))
        rem = jnp.float32(CAP) - n_gt
        eqc = eq1.astype(jnp.float32) + eq2.astype(jnp.float32)
        ex_eq = _excl_prefix_flat(eqc)
        kept1 = gt1 | (eq1 & (ex_eq < rem))
        kept2 = gt2 | (eq2 & ((ex_eq + eq1.astype(jnp.float32)) < rem))

        keptc = kept1.astype(jnp.float32) + kept2.astype(jnp.float32)
        ex_k = _excl_prefix_flat(keptc)
        pos1 = ex_k.astype(jnp.int32)
        pos2 = (ex_k + kept1.astype(jnp.float32)).astype(jnp.int32)

        dest1 = jnp.where(kept1, e * CAP + pos1, dest1)
        dest2 = jnp.where(kept2, e * CAP + pos2, dest2)
        cidx1 = jnp.where(kept1, e * CAP + pos1, cidx1)
        cidx2 = jnp.where(kept2, e * CAP + pos2, cidx2)
        g1 = jnp.where(kept1, v1r, g1)
        g2 = jnp.where(kept2, v2r, g2)

    dest_ref[...] = jnp.stack([dest1.reshape(SEQ), dest2.reshape(SEQ)])
    cidx_ref[...] = jnp.stack([cidx1.reshape(SEQ), cidx2.reshape(SEQ)])
    g_ref[...] = jnp.stack([g1.reshape(SEQ), g2.reshape(SEQ)]).reshape(
        TOP_K, 1, SEQ)

    importance = jnp.sum(scores, axis=0)
    load_v = jnp.stack(load)
    aux = jnp.sum(importance * load_v) * (NUM_E / (SEQ * SEQ))
    aux_ref[...] = aux.reshape(1, 1)


_SC_INFO = plsc.get_sparse_core_info()
_NC = _SC_INFO.num_cores
_NSUB = _SC_INFO.num_subcores
_NW = _NC * _NSUB
_RPW = NROWS // _NW
_GPW = (SEQ * TOP_K) // _NW
_CH = 64


_ZW = NTOK // _NSUB
_DR = (SEQ * TOP_K) // (_NSUB * _C)


def _dispatch_body(x_hbm, destf_hbm, xbuf_hbm,
                   didx_v, dval_v, zero_v, shared_tok, idx_c, rows_v, sem):
    cid = lax.axis_index("c")
    sid = lax.axis_index("s")
    wid = sid * _NC + cid

    def zf(i, c):
        zero_v[pl.ds(i * 16, 16)] = jnp.zeros((16,), jnp.int32)
        return c

    lax.fori_loop(0, _ZW // 16, zf, 0)
    pltpu.sync_copy(zero_v, shared_tok.at[pl.ds(sid * _ZW, _ZW)])
    plsc.subcore_barrier()

    pltpu.sync_copy(destf_hbm.at[pl.ds(sid * _DR, _DR)], didx_v)
    for r in range(_DR):
        rowbase = lax.rem(sid * _DR + r, _R) * _C
        for c16 in range(_C // 16):
            dval_v[r, pl.ds(c16 * 16, 16)] = (
                rowbase + c16 * 16 + lax.broadcasted_iota(jnp.int32, (16,), 0))
    for r in range(_DR):
        pltpu.sync_copy(dval_v.at[r], shared_tok.at[didx_v.at[r]], add=True)
    plsc.subcore_barrier()

    base = wid * _RPW
    for ch in range(_RPW // _CH):
        off = base + ch * _CH
        pltpu.sync_copy(shared_tok.at[pl.ds(off, _CH)], idx_c)
        pltpu.async_copy(x_hbm.at[idx_c], rows_v, sem).wait()
        pltpu.sync_copy(rows_v, xbuf_hbm.at[pl.ds(off, _CH)])


def _gatherout_body(ybuf_hbm, cidxf_hbm, yg_hbm, idx_c, rows_v, sem):
    cid = lax.axis_index("c")
    sid = lax.axis_index("s")
    wid = sid * _NC + cid
    base = wid * _GPW
    for ch in range(_GPW // _CH):
        off = base + ch * _CH
        pltpu.sync_copy(cidxf_hbm.at[pl.ds(off, _CH)], idx_c)
        pltpu.async_copy(ybuf_hbm.at[idx_c], rows_v, sem).wait()
        pltpu.sync_copy(rows_v, yg_hbm.at[pl.ds(off, _CH)])


def _mlp_body(xb_ref, w1_ref, w2_ref, out_ref):
    xb = xb_ref[...].astype(jnp.bfloat16)
    w1 = w1_ref[0].astype(jnp.bfloat16)
    w2 = w2_ref[0].astype(jnp.bfloat16)
    h = lax.dot_general(xb, w1, (((1,), (1,)), ((), ())),
                        preferred_element_type=jnp.float32)
    h = 0.5 * h * (1.0 + lax.erf(h * 0.7071067811865476))
    y = lax.dot_general(h.astype(jnp.bfloat16), w2, (((1,), (1,)), ((), ())),
                        preferred_element_type=jnp.float32)
    out_ref[...] = y


def _combine_body(yg_ref, g_ref, out_ref):
    y = yg_ref[...]
    g = g_ref[...]
    out_ref[...] = g[0, 0, :, None] * y[0] + g[1, 0, :, None] * y[1]


_BS = 640
_CBS = 512


@jax.jit
def kernel(x, Wg, W1, W2):
    scores = pl.pallas_call(
        _score_body,
        grid=(SEQ // _CBS,),
        out_shape=jax.ShapeDtypeStruct((SEQ, NUM_E), jnp.float32),
        in_specs=[
            pl.BlockSpec((_CBS, D_MODEL), lambda s: (s, 0)),
            pl.BlockSpec((NUM_E, D_MODEL), lambda s: (0, 0)),
        ],
        out_specs=pl.BlockSpec((_CBS, NUM_E), lambda s: (s, 0)),
    )(x, Wg)

    dest, cidx, g, aux = pl.pallas_call(
        _route_body,
        out_shape=(
            jax.ShapeDtypeStruct((TOP_K, SEQ), jnp.int32),
            jax.ShapeDtypeStruct((TOP_K, SEQ), jnp.int32),
            jax.ShapeDtypeStruct((TOP_K, 1, SEQ), jnp.float32),
            jax.ShapeDtypeStruct((1, 1), jnp.float32),
        ),
        in_specs=[
            pl.BlockSpec((SEQ, NUM_E), lambda: (0, 0)),
        ],
        out_specs=(
            pl.BlockSpec((TOP_K, SEQ), lambda: (0, 0)),
            pl.BlockSpec((TOP_K, SEQ), lambda: (0, 0)),
            pl.BlockSpec((TOP_K, 1, SEQ), lambda: (0, 0, 0)),
            pl.BlockSpec((1, 1), lambda: (0, 0)),
        ),
    )(scores)

    mesh = plsc.VectorSubcoreMesh(core_axis_name="c", subcore_axis_name="s")

    sc_params = pltpu.CompilerParams(needs_layout_passes=False)
    xbuf = pl.kernel(
        _dispatch_body,
        out_type=jax.ShapeDtypeStruct((NROWS, D_MODEL), jnp.float32),
        mesh=mesh,
        compiler_params=sc_params,
        scratch_types=[
            pltpu.VMEM((_DR, _C), jnp.int32),
            pltpu.VMEM((_DR, _C), jnp.int32),
            pltpu.VMEM((_ZW,), jnp.int32),
            pltpu.VMEM_SHARED((NTOK,), jnp.int32),
            pltpu.VMEM((_CH,), jnp.int32),
            pltpu.VMEM((_CH, D_MODEL), jnp.float32),
            pltpu.SemaphoreType.DMA,
        ],
    )(x, dest.reshape((SEQ * TOP_K) // _C, _C))

    ybuf = pl.pallas_call(
        _mlp_body,
        grid=(NUM_E, CAP // _BS),
        out_shape=jax.ShapeDtypeStruct((NROWS, D_MODEL), jnp.float32),
        in_specs=[
            pl.BlockSpec((_BS, D_MODEL), lambda e, t: (e * (CAP // _BS) + t, 0)),
            pl.BlockSpec((1, HIDDEN, D_MODEL), lambda e, t: (e, 0, 0)),
            pl.BlockSpec((1, D_MODEL, HIDDEN), lambda e, t: (e, 0, 0)),
        ],
        out_specs=pl.BlockSpec((_BS, D_MODEL),
                               lambda e, t: (e * (CAP // _BS) + t, 0)),
    )(xbuf, W1, W2)

    yg = pl.kernel(
        _gatherout_body,
        out_type=jax.ShapeDtypeStruct((SEQ * TOP_K, D_MODEL), jnp.float32),
        mesh=mesh,
        compiler_params=sc_params,
        scratch_types=[
            pltpu.VMEM((_CH,), jnp.int32),
            pltpu.VMEM((_CH, D_MODEL), jnp.float32),
            pltpu.SemaphoreType.DMA,
        ],
    )(ybuf, cidx.reshape(SEQ * TOP_K))

    out = pl.pallas_call(
        _combine_body,
        grid=(SEQ // _CBS,),
        out_shape=jax.ShapeDtypeStruct((SEQ, D_MODEL), jnp.float32),
        in_specs=[
            pl.BlockSpec((TOP_K, _CBS, D_MODEL), lambda s: (0, s, 0)),
            pl.BlockSpec((TOP_K, 1, _CBS), lambda s: (0, 0, s)),
        ],
        out_specs=pl.BlockSpec((_CBS, D_MODEL), lambda s: (s, 0)),
    )(yg.reshape(TOP_K, SEQ, D_MODEL), g)

    return out, aux.reshape(())
